# baseline (speedup 1.0000x reference)
import jax
import jax.numpy as jnp
import numpy as np
from jax.experimental import pallas as pl
from jax.experimental.pallas import tpu as pltpu

_ANCH = (
    ((10.0, 13.0), (16.0, 30.0), (33.0, 23.0)),
    ((30.0, 61.0), (62.0, 45.0), (59.0, 119.0)),
    ((116.0, 90.0), (156.0, 198.0), (373.0, 326.0)),
)
_W = (76, 38, 19)
_H = (76, 38, 19)
_HC = (8, 8, 19)
_NCH = (10, 5, 1)
_J0 = (0, 10, 15)
_ROW0 = (0, 17328, 21660)
_STRIDE = (8.0, 16.0, 32.0)
_ROWS = 22743


def _mul_const(s, a):
    m = np.ones((1, 85), np.float32)
    m[0, 0] = m[0, 1] = _STRIDE[s]
    m[0, 2] = _ANCH[s][a][0]
    m[0, 3] = _ANCH[s][a][1]
    return m


_MUL_TAB = np.concatenate(
    [_mul_const(s, a) for s in range(3) for a in range(3)], 0
).reshape(9, 1, 85)


def _body(x0_ref, x1_ref, x2_ref, pre_ref, mul_ref, out_ref):
    j = pl.program_id(1)
    pre = pre_ref[...]

    for s, x_ref in enumerate((x0_ref, x1_ref, x2_ref)):
        jlo, nch, w, hc = _J0[s], _NCH[s], _W[s], _HC[s]
        st = _STRIDE[s]

        @pl.when((j >= jlo) & (j < jlo + nch))
        def _(s=s, x_ref=x_ref, jlo=jlo, nch=nch, w=w, hc=hc, st=st):
            k = j - jlo
            lane = jax.lax.broadcasted_iota(jnp.int32, (w, 85), 1)
            isexp = (lane == 2) | (lane == 3)
            wv = jax.lax.broadcasted_iota(
            jnp.int32, (w, 85), 0).astype(jnp.float32) * st
            addx = jnp.where(lane == 0, wv, 0.0)
            kf = k.astype(jnp.float32)
            outs = []
            for h in range(hc):
                t = jnp.transpose(x_ref[:, h, :])
                hy = (kf * hc + h) * st
                add = addx + jnp.where(lane == 1, hy, 0.0)
                row = []
                for a in range(3):
                    u = t[:, a * 85:(a + 1) * 85] * pre
                    e = jnp.exp(u)
                    nl = jnp.where(isexp, e, 1.0 / (1.0 + e))
                    row.append(nl * mul_ref[s * 3 + a] + add)
                outs.append(row)
            base = _ROW0[s]
            for kk in range(nch):
                realh = min(_H[s] - kk * hc, hc)

                @pl.when(k == kk)
                def _(kk=kk, realh=realh):
                    for h in range(realh):
                        r0 = base + 3 * w * (kk * hc + h)
                        for a in range(3):
                            out_ref[0, r0 + a:r0 + 3 * w:3, :] = outs[h][a]


def kernel(x0, x1, x2, idf_logits):
    bs = x0.shape[0]
    xv = [x.reshape(bs * 255, _H[s], _W[s]) for s, x in enumerate((x0, x1, x2))]
    pre = jnp.concatenate(
        [jnp.asarray([-1.0, -1.0, 1.0, 1.0, -1.0], jnp.float32),
         -idf_logits]).reshape(1, 85)

    return pl.pallas_call(
        _body,
        grid=(bs, 16),
        in_specs=[
            pl.BlockSpec((255, 8, 76), lambda b, j: (b, jnp.minimum(j, 9), 0)),
            pl.BlockSpec((255, 8, 38), lambda b, j: (b, jnp.clip(j - 10, 0, 4), 0)),
            pl.BlockSpec((255, 19, 19), lambda b, j: (b, 0, 0)),
            pl.BlockSpec((1, 85), lambda b, j: (0, 0)),
            pl.BlockSpec((9, 1, 85), lambda b, j: (0, 0, 0)),
        ],
        out_specs=pl.BlockSpec((1, _ROWS, 85), lambda b, j: (b, 0, 0)),
        out_shape=jax.ShapeDtypeStruct((bs, _ROWS, 85), jnp.float32),
        compiler_params=pltpu.CompilerParams(
            dimension_semantics=("parallel", "arbitrary"),
            vmem_limit_bytes=100 * 2**20),
    )(xv[0], xv[1], xv[2], pre, jnp.asarray(_MUL_TAB))


# speedup vs baseline: 1.7628x; 1.7628x over previous
import jax
import jax.numpy as jnp
import numpy as np
from jax.experimental import pallas as pl
from jax.experimental.pallas import tpu as pltpu

_ANCH = (
    ((10.0, 13.0), (16.0, 30.0), (33.0, 23.0)),
    ((30.0, 61.0), (62.0, 45.0), (59.0, 119.0)),
    ((116.0, 90.0), (156.0, 198.0), (373.0, 326.0)),
)
_W = (76, 38, 19)
_HW = (5776, 1444, 361)
_COLS = (768, 768, 384)
_NCH = (8, 2, 1)
_HWP = tuple(c * n for c, n in zip(_COLS, _NCH))
_J0 = (0, 8, 10)
_ROW0 = (0, 17328, 21660)
_STRIDE = (8.0, 16.0, 32.0)
_ROWS = 22743
_TABROWS = 768


def _build_add_table():
    blocks = []
    for s in range(3):
        w, st = _W[s], _STRIDE[s]
        t = np.zeros((_NCH[s], _TABROWS, 85), np.float32)
        hw = np.arange(_COLS[s])
        for k in range(_NCH[s]):
            g = hw + k * _COLS[s]
            t[k, :_COLS[s], 0] = (g % w) * st
            t[k, :_COLS[s], 1] = (g // w) * st
        blocks.append(t)
    return np.concatenate(blocks, 0)


_ADD_TAB = _build_add_table()


def _mul_const(s, a):
    m = np.ones((1, 85), np.float32)
    m[0, 0] = m[0, 1] = _STRIDE[s]
    m[0, 2] = _ANCH[s][a][0]
    m[0, 3] = _ANCH[s][a][1]
    return m


_MUL_TAB = np.concatenate(
    [_mul_const(s, a) for s in range(3) for a in range(3)], 0
).reshape(9, 1, 85)


def _body(x0_ref, x1_ref, x2_ref, add_ref, pre_ref, mul_ref, out_ref, scr_ref):
    j = pl.program_id(1)
    pre = pre_ref[...]
    lane = jax.lax.broadcasted_iota(jnp.int32, (128, 85), 1)
    isexp = (lane == 2) | (lane == 3)

    for s, x_ref in enumerate((x0_ref, x1_ref, x2_ref)):
        jlo, nch, cols = _J0[s], _NCH[s], _COLS[s]

        @pl.when((j >= jlo) & (j < jlo + nch))
        def _(s=s, x_ref=x_ref, jlo=jlo, nch=nch, cols=cols):
            for c in range(cols // 128):
                t = jnp.transpose(x_ref[0, :, c * 128:(c + 1) * 128])
                add = add_ref[0, c * 128:(c + 1) * 128, :]
                for a in range(3):
                    u = t[:, a * 85:(a + 1) * 85] * pre
                    e = jnp.exp(u)
                    nl = jnp.where(isexp, e, 1.0 / (1.0 + e))
                    o = nl * mul_ref[s * 3 + a] + add
                    s0 = 3 * 128 * c + a
                    scr_ref[s0:s0 + 3 * 128:3, :] = o
            k = j - jlo
            base = _ROW0[s]
            rows = 3 * cols
            tail_rows = (_HW[s] - (nch - 1) * cols) * 3
            if nch == 1:
                out_ref[0, base:base + tail_rows, :] = scr_ref[:tail_rows, :]
            else:
                @pl.when(k < nch - 1)
                def _():
                    out_ref[0, pl.ds(base + k * rows, rows), :] = \
                        scr_ref[:rows, :]

                @pl.when(k == nch - 1)
                def _():
                    r0 = base + (nch - 1) * rows
                    out_ref[0, r0:base + _HW[s] * 3, :] = scr_ref[:tail_rows, :]


def kernel(x0, x1, x2, idf_logits):
    bs = x0.shape[0]
    xp = []
    for x, hw, hwp in zip((x0, x1, x2), _HW, _HWP):
        xr = x.reshape(bs, 255, hw)
        xp.append(jnp.pad(xr, ((0, 0), (0, 0), (0, hwp - hw))))
    pre = jnp.concatenate(
        [jnp.asarray([-1.0, -1.0, 1.0, 1.0, -1.0], jnp.float32),
         -idf_logits]).reshape(1, 85)
    add_tab = jnp.asarray(_ADD_TAB)

    return pl.pallas_call(
        _body,
        grid=(bs, 11),
        in_specs=[
            pl.BlockSpec((1, 255, 768), lambda b, j: (b, 0, jnp.minimum(j, 7))),
            pl.BlockSpec((1, 255, 768), lambda b, j: (b, 0, jnp.clip(j - 8, 0, 1))),
            pl.BlockSpec((1, 255, 384), lambda b, j: (b, 0, 0)),
            pl.BlockSpec((1, _TABROWS, 85), lambda b, j: (j, 0, 0)),
            pl.BlockSpec((1, 85), lambda b, j: (0, 0)),
            pl.BlockSpec((9, 1, 85), lambda b, j: (0, 0, 0)),
        ],
        out_specs=pl.BlockSpec((1, _ROWS, 85), lambda b, j: (b, 0, 0)),
        out_shape=jax.ShapeDtypeStruct((bs, _ROWS, 85), jnp.float32),
        scratch_shapes=[pltpu.VMEM((3 * 768, 85), jnp.float32)],
        compiler_params=pltpu.CompilerParams(
            dimension_semantics=("parallel", "arbitrary"),
            vmem_limit_bytes=100 * 2**20),
    )(xp[0], xp[1], xp[2], add_tab, pre, jnp.asarray(_MUL_TAB))


# speedup vs baseline: 2.1343x; 1.2108x over previous
import jax
import jax.numpy as jnp
import numpy as np
from jax.experimental import pallas as pl
from jax.experimental.pallas import tpu as pltpu

_ANCH = (
    ((10.0, 13.0), (16.0, 30.0), (33.0, 23.0)),
    ((30.0, 61.0), (62.0, 45.0), (59.0, 119.0)),
    ((116.0, 90.0), (156.0, 198.0), (373.0, 326.0)),
)
_W = (76, 38, 19)
_HW = (5776, 1444, 361)
_ROW0 = (0, 17328, 21660)
_TAB0 = (0, 5776, 7220)
_STRIDE = (8.0, 16.0, 32.0)
_ROWS = 22743


def _build_add_table():
    blocks = []
    for s in range(3):
        w, st = _W[s], _STRIDE[s]
        hw = np.arange(_HW[s])
        t = np.zeros((_HW[s], 85), np.float32)
        t[:, 0] = (hw % w) * st
        t[:, 1] = (hw // w) * st
        blocks.append(t)
    return np.concatenate(blocks, 0)


_ADD_TAB = _build_add_table()


def _mul_const(s, a):
    m = np.ones((1, 85), np.float32)
    m[0, 0] = m[0, 1] = _STRIDE[s]
    m[0, 2] = _ANCH[s][a][0]
    m[0, 3] = _ANCH[s][a][1]
    return m


_MUL_TAB = np.concatenate(
    [_mul_const(s, a) for s in range(3) for a in range(3)], 0
).reshape(9, 1, 85)


def _body(x0_ref, x1_ref, x2_ref, add_ref, pre_ref, mul_ref, out_ref):
    j = pl.program_id(1)
    pre = pre_ref[...]

    for s, x_ref in enumerate((x0_ref, x1_ref, x2_ref)):

        @pl.when(j == s)
        def _(s=s, x_ref=x_ref):
            base, tab0, hw = _ROW0[s], _TAB0[s], _HW[s]
            nsub = (hw + 127) // 128
            for c in range(nsub):
                c0 = c * 128
                cw = min(128, hw - c0)
                t = jnp.transpose(x_ref[0, :, c0:c0 + cw])
                add = add_ref[tab0 + c0:tab0 + c0 + cw, :]
                lane = jax.lax.broadcasted_iota(jnp.int32, (cw, 85), 1)
                isexp = (lane == 2) | (lane == 3)
                for a in range(3):
                    u = t[:, a * 85:(a + 1) * 85] * pre
                    e = jnp.exp(u)
                    nl = jnp.where(isexp, e, 1.0 / (1.0 + e))
                    o = nl * mul_ref[s * 3 + a] + add
                    r0 = base + 3 * c0 + a
                    out_ref[0, r0:r0 + 3 * cw:3, :] = o


def kernel(x0, x1, x2, idf_logits):
    bs = x0.shape[0]
    xr = [x.reshape(bs, 255, hw) for x, hw in zip((x0, x1, x2), _HW)]
    pre = jnp.concatenate(
        [jnp.asarray([-1.0, -1.0, 1.0, 1.0, -1.0], jnp.float32),
         -idf_logits]).reshape(1, 85)

    return pl.pallas_call(
        _body,
        grid=(bs, 3),
        in_specs=[
            pl.BlockSpec((1, 255, 5776), lambda b, j: (b, 0, 0)),
            pl.BlockSpec((1, 255, 1444), lambda b, j: (b, 0, 0)),
            pl.BlockSpec((1, 255, 361), lambda b, j: (b, 0, 0)),
            pl.BlockSpec((7581, 85), lambda b, j: (0, 0)),
            pl.BlockSpec((1, 85), lambda b, j: (0, 0)),
            pl.BlockSpec((9, 1, 85), lambda b, j: (0, 0, 0)),
        ],
        out_specs=pl.BlockSpec((1, _ROWS, 85), lambda b, j: (b, 0, 0)),
        out_shape=jax.ShapeDtypeStruct((bs, _ROWS, 85), jnp.float32),
        compiler_params=pltpu.CompilerParams(
            dimension_semantics=("parallel", "arbitrary"),
            vmem_limit_bytes=100 * 2**20),
    )(xr[0], xr[1], xr[2], jnp.asarray(_ADD_TAB), pre, jnp.asarray(_MUL_TAB))
